# Initial kernel scaffold; baseline (speedup 1.0000x reference)
import jax
import jax.numpy as jnp
from jax.experimental import pallas as pl


def kernel(x, edge_index, e, W_node, b_node, W_edge, b_edge, W_gcn0, b_gcn0, W_gcn1, b_gcn1, W_gcn2, b_gcn2, W_src, W_dst, w_out, b_out):
    raise NotImplementedError("write your pallas kernel here")


# speedup vs baseline: 4.4789x; 4.4789x over previous
import functools

import jax
import jax.numpy as jnp
from jax import lax
from jax.experimental import pallas as pl
from jax.experimental.pallas import tpu as pltpu
from jax.experimental.pallas import tpu_sc as plsc

N = 10000
E = 320000
D = 128
DE = 16
NPAD = 10240
NC = 2
NS = 16
NW = NC * NS
EPT = E // NW
CHUNK = 128
NFULL = EPT // CHUNK
TAIL = EPT - NFULL * CHUNK
RPT = NPAD // NS
WDEG = 128


def _sc_mesh():
    return plsc.VectorSubcoreMesh(core_axis_name="c", subcore_axis_name="s")


def _deg_body(dst_hbm, ones_hbm, zeros_hbm, out_hbm, didx, didx_t, ones_v,
              deg_sh):
    core = lax.axis_index("c")
    sub = lax.axis_index("s")
    base = (core * NS + sub) * EPT
    r0 = sub * RPT
    pltpu.sync_copy(zeros_hbm.at[pl.ds(r0, RPT)], deg_sh.at[pl.ds(r0, RPT)])
    pltpu.sync_copy(ones_hbm, ones_v)
    plsc.subcore_barrier()

    @pl.loop(0, NFULL)
    def _(c):
        off = base + c * CHUNK
        pltpu.sync_copy(dst_hbm.at[pl.ds(off, CHUNK)], didx)
        pltpu.sync_copy(ones_v, deg_sh.at[didx], add=True)

    offt = base + NFULL * CHUNK
    pltpu.sync_copy(dst_hbm.at[pl.ds(offt, TAIL)], didx_t)
    pltpu.sync_copy(ones_v.at[pl.ds(0, TAIL)], deg_sh.at[didx_t], add=True)
    plsc.subcore_barrier()
    pltpu.sync_copy(deg_sh.at[pl.ds(r0, RPT)],
                    out_hbm.at[core, pl.ds(r0, RPT)])


def _deg(dst, ones16, zeros16):
    return pl.kernel(
        _deg_body,
        out_type=jax.ShapeDtypeStruct((NC, NPAD, WDEG), jnp.float32),
        mesh=_sc_mesh(),
        scratch_types=[
            pltpu.VMEM((CHUNK,), jnp.int32),
            pltpu.VMEM((TAIL,), jnp.int32),
            pltpu.VMEM((CHUNK, WDEG), jnp.float32),
            pltpu.VMEM_SHARED((NPAD, WDEG), jnp.float32),
        ],
    )(dst, ones16, zeros16)


def _scat_body(h_hbm, src_hbm, dst_hbm, zeros_hbm, out_hbm,
               sidx, didx, sidx_t, didx_t, rows, rows_t, agg_sh, sem):
    core = lax.axis_index("c")
    sub = lax.axis_index("s")
    base = (core * NS + sub) * EPT
    r0 = sub * RPT
    pltpu.sync_copy(zeros_hbm.at[pl.ds(r0, RPT)], agg_sh.at[pl.ds(r0, RPT)])
    plsc.subcore_barrier()

    @pl.loop(0, NFULL)
    def _(c):
        off = base + c * CHUNK
        pltpu.sync_copy(src_hbm.at[pl.ds(off, CHUNK)], sidx)
        pltpu.async_copy(h_hbm.at[sidx], rows, sem).wait()
        pltpu.sync_copy(dst_hbm.at[pl.ds(off, CHUNK)], didx)
        pltpu.sync_copy(rows, agg_sh.at[didx], add=True)

    offt = base + NFULL * CHUNK
    pltpu.sync_copy(src_hbm.at[pl.ds(offt, TAIL)], sidx_t)
    pltpu.async_copy(h_hbm.at[sidx_t], rows_t, sem).wait()
    pltpu.sync_copy(dst_hbm.at[pl.ds(offt, TAIL)], didx_t)
    pltpu.sync_copy(rows_t, agg_sh.at[didx_t], add=True)
    plsc.subcore_barrier()
    pltpu.sync_copy(agg_sh.at[pl.ds(r0, RPT)],
                    out_hbm.at[core, pl.ds(r0, RPT)])


def _scatter(h, src, dst, zeros128):
    return pl.kernel(
        _scat_body,
        out_type=jax.ShapeDtypeStruct((NC, NPAD, D), jnp.float32),
        mesh=_sc_mesh(),
        scratch_types=[
            pltpu.VMEM((CHUNK,), jnp.int32),
            pltpu.VMEM((CHUNK,), jnp.int32),
            pltpu.VMEM((TAIL,), jnp.int32),
            pltpu.VMEM((TAIL,), jnp.int32),
            pltpu.VMEM((CHUNK, D), jnp.float32),
            pltpu.VMEM((TAIL, D), jnp.float32),
            pltpu.VMEM_SHARED((NPAD, D), jnp.float32),
            pltpu.SemaphoreType.DMA,
        ],
    )(h, src, dst, zeros128)


def _v_add(r1, r2, nrows):
    @pl.loop(0, nrows)
    def _(r):
        @pl.loop(0, D // 16, unroll=8)
        def _(q):
            sl = pl.ds(q * 16, 16)
            plsc.addupdate(r1.at[r, sl], r2[r, sl])


def _v_body(asrc_hbm, adst_hbm, src_hbm, dst_hbm, out_hbm,
            sidx, didx, sidx_t, didx_t, r1, r2, r1t, r2t, sem1, sem2):
    core = lax.axis_index("c")
    sub = lax.axis_index("s")
    base = (core * NS + sub) * EPT

    @pl.loop(0, NFULL)
    def _(c):
        off = base + c * CHUNK
        pltpu.sync_copy(src_hbm.at[pl.ds(off, CHUNK)], sidx)
        pltpu.sync_copy(dst_hbm.at[pl.ds(off, CHUNK)], didx)
        g1 = pltpu.async_copy(asrc_hbm.at[sidx], r1, sem1)
        g2 = pltpu.async_copy(adst_hbm.at[didx], r2, sem2)
        g1.wait()
        g2.wait()
        _v_add(r1, r2, CHUNK)
        pltpu.sync_copy(r1, out_hbm.at[pl.ds(off, CHUNK)])

    offt = base + NFULL * CHUNK
    pltpu.sync_copy(src_hbm.at[pl.ds(offt, TAIL)], sidx_t)
    pltpu.sync_copy(dst_hbm.at[pl.ds(offt, TAIL)], didx_t)
    g1 = pltpu.async_copy(asrc_hbm.at[sidx_t], r1t, sem1)
    g2 = pltpu.async_copy(adst_hbm.at[didx_t], r2t, sem2)
    g1.wait()
    g2.wait()
    _v_add(r1t, r2t, TAIL)
    pltpu.sync_copy(r1t, out_hbm.at[pl.ds(offt, TAIL)])


def _vkern(a_src, a_dst, src, dst):
    return pl.kernel(
        _v_body,
        out_type=jax.ShapeDtypeStruct((E, D), jnp.float32),
        mesh=_sc_mesh(),
        scratch_types=[
            pltpu.VMEM((CHUNK,), jnp.int32),
            pltpu.VMEM((CHUNK,), jnp.int32),
            pltpu.VMEM((TAIL,), jnp.int32),
            pltpu.VMEM((TAIL,), jnp.int32),
            pltpu.VMEM((CHUNK, D), jnp.float32),
            pltpu.VMEM((CHUNK, D), jnp.float32),
            pltpu.VMEM((TAIL, D), jnp.float32),
            pltpu.VMEM((TAIL, D), jnp.float32),
            pltpu.SemaphoreType.DMA,
            pltpu.SemaphoreType.DMA,
        ],
    )(a_src, a_dst, src, dst)


BM = 512
BE = 4000


def _enc_body(x_ref, w_ref, b_ref, o_ref):
    o_ref[...] = jnp.dot(x_ref[...], w_ref[...],
                         preferred_element_type=jnp.float32) + b_ref[...]


def _encode(x_pad, W, b_row):
    return pl.pallas_call(
        _enc_body,
        grid=(NPAD // BM,),
        in_specs=[
            pl.BlockSpec((BM, D), lambda i: (i, 0)),
            pl.BlockSpec((D, D), lambda i: (0, 0)),
            pl.BlockSpec((1, D), lambda i: (0, 0)),
        ],
        out_specs=pl.BlockSpec((BM, D), lambda i: (i, 0)),
        out_shape=jax.ShapeDtypeStruct((NPAD, D), jnp.float32),
    )(x_pad, W, b_row)


def _norm_agg(a_ref, d_ref):
    agg = a_ref[0] + a_ref[1]
    deg = d_ref[0, :, 0:1] + d_ref[1, :, 0:1]
    return agg * (1.0 / jnp.maximum(deg, 1.0))


def _layer_body(a_ref, d_ref, w_ref, b_ref, o_ref):
    z = _norm_agg(a_ref, d_ref)
    o_ref[...] = jax.nn.relu(
        jnp.dot(z, w_ref[...], preferred_element_type=jnp.float32)
        + b_ref[...])


def _layer(agg2, deg2, W, b_row):
    return pl.pallas_call(
        _layer_body,
        grid=(NPAD // BM,),
        in_specs=[
            pl.BlockSpec((NC, BM, D), lambda i: (0, i, 0)),
            pl.BlockSpec((NC, BM, WDEG), lambda i: (0, i, 0)),
            pl.BlockSpec((D, D), lambda i: (0, 0)),
            pl.BlockSpec((1, D), lambda i: (0, 0)),
        ],
        out_specs=pl.BlockSpec((BM, D), lambda i: (i, 0)),
        out_shape=jax.ShapeDtypeStruct((NPAD, D), jnp.float32),
    )(agg2, deg2, W, b_row)


def _layer3_body(a_ref, d_ref, w_ref, b_ref, ws_ref, wd_ref, os_ref, od_ref):
    z = _norm_agg(a_ref, d_ref)
    h = jax.nn.relu(
        jnp.dot(z, w_ref[...], preferred_element_type=jnp.float32)
        + b_ref[...])
    os_ref[...] = jnp.dot(h, ws_ref[...], preferred_element_type=jnp.float32)
    od_ref[...] = jnp.dot(h, wd_ref[...], preferred_element_type=jnp.float32)


def _layer3(agg2, deg2, W, b_row, W_src, W_dst):
    return pl.pallas_call(
        _layer3_body,
        grid=(NPAD // BM,),
        in_specs=[
            pl.BlockSpec((NC, BM, D), lambda i: (0, i, 0)),
            pl.BlockSpec((NC, BM, WDEG), lambda i: (0, i, 0)),
            pl.BlockSpec((D, D), lambda i: (0, 0)),
            pl.BlockSpec((1, D), lambda i: (0, 0)),
            pl.BlockSpec((D, D), lambda i: (0, 0)),
            pl.BlockSpec((D, D), lambda i: (0, 0)),
        ],
        out_specs=[
            pl.BlockSpec((BM, D), lambda i: (i, 0)),
            pl.BlockSpec((BM, D), lambda i: (i, 0)),
        ],
        out_shape=[
            jax.ShapeDtypeStruct((NPAD, D), jnp.float32),
            jax.ShapeDtypeStruct((NPAD, D), jnp.float32),
        ],
    )(agg2, deg2, W, b_row, W_src, W_dst)


def _score_body(v_ref, e_ref, we_ref, be_ref, wo_ref, bo_ref, o_ref):
    eh = jnp.dot(e_ref[...], we_ref[...],
                 preferred_element_type=jnp.float32) + be_ref[...]
    t = jax.nn.relu(v_ref[...] + eh)
    o_ref[...] = jnp.sum(t * wo_ref[...], axis=1, keepdims=True) + bo_ref[...]


def _score(V, e, W_edge, be_row, wo_row, bo_11):
    return pl.pallas_call(
        _score_body,
        grid=(E // BE,),
        in_specs=[
            pl.BlockSpec((BE, D), lambda i: (i, 0)),
            pl.BlockSpec((BE, DE), lambda i: (i, 0)),
            pl.BlockSpec((DE, D), lambda i: (0, 0)),
            pl.BlockSpec((1, D), lambda i: (0, 0)),
            pl.BlockSpec((1, D), lambda i: (0, 0)),
            pl.BlockSpec((1, 1), lambda i: (0, 0)),
        ],
        out_specs=pl.BlockSpec((BE, 1), lambda i: (i, 0)),
        out_shape=jax.ShapeDtypeStruct((E, 1), jnp.float32),
    )(V, e, W_edge, be_row, wo_row, bo_11)


def kernel(x, edge_index, e, W_node, b_node, W_edge, b_edge,
           W_gcn0, b_gcn0, W_gcn1, b_gcn1, W_gcn2, b_gcn2,
           W_src, W_dst, w_out, b_out):
    src = edge_index[0].astype(jnp.int32)
    dst = edge_index[1].astype(jnp.int32)
    x_pad = jnp.pad(x, ((0, NPAD - N), (0, 0)))
    zeros128 = jnp.zeros((NPAD, D), jnp.float32)
    zeros16 = jnp.zeros((NPAD, WDEG), jnp.float32)
    ones16 = jnp.ones((CHUNK, WDEG), jnp.float32)

    h = _encode(x_pad, W_node, b_node.reshape(1, D))
    deg2 = _deg(dst, ones16, zeros16)
    for W, b in ((W_gcn0, b_gcn0), (W_gcn1, b_gcn1)):
        agg2 = _scatter(h, src, dst, zeros128)
        h = _layer(agg2, deg2, W, b.reshape(1, D))
    agg2 = _scatter(h, src, dst, zeros128)
    a_src, a_dst = _layer3(agg2, deg2, W_gcn2, b_gcn2.reshape(1, D),
                           W_src, W_dst)
    V = _vkern(a_src, a_dst, src, dst)
    scores = _score(V, e, W_edge, b_edge.reshape(1, D),
                    w_out.reshape(1, D), b_out.reshape(1, 1))
    return scores[:, 0]
